# baseline (speedup 1.0000x reference)
import functools

import jax
import jax.numpy as jnp
from jax import lax
from jax.experimental import pallas as pl
from jax.experimental.pallas import tpu as pltpu, tpu_sc as plsc

_N = 10000
_NODE_MASKING = 0.3
_RADIUS = 0.75
_K = 50
_NUM_BINS = 5
_CUTOFF = 5.0
_N_KEEP = int(_N * (1.0 - _NODE_MASKING))

_BR = 1024
_PAD = 7168
_KPAD = 64
_F = 128

_E = _N_KEEP * _K
_NW = 32
_EPAD = 350208
_BPW = _EPAD // _NW


def _augment_consts():
    base = jax.random.key(1)
    k1 = jax.random.fold_in(base, 0)
    k2 = jax.random.fold_in(base, 1)
    k3 = jax.random.fold_in(base, 2)
    scores = jax.random.uniform(k1, (_N,))
    keep_idx = jnp.argsort(scores)[:_N_KEEP]
    dirs = jax.random.normal(k2, (_N_KEEP, 3), dtype=jnp.float32)
    dirs = dirs / (jnp.linalg.norm(dirs, axis=1, keepdims=True) + 1e-12)
    u = jax.random.uniform(k3, (_N_KEEP, 1), dtype=jnp.float32)
    noise = dirs * _RADIUS * (u ** (1.0 / 3.0))
    return keep_idx, noise


def _knn_kernel(a_ref, btb_ref, sqc_ref, nbr_ref, sel_ref):
    i = pl.program_id(0)
    a = a_ref[...]
    sq_r = jnp.sum(a * a, axis=1, keepdims=True)
    dot = jnp.dot(a.astype(jnp.bfloat16), btb_ref[...],
                  preferred_element_type=jnp.float32)
    d2s = sq_r + sqc_ref[0:1, :] - 2.0 * dot
    col = jax.lax.broadcasted_iota(jnp.int32, d2s.shape, 1)
    row = jax.lax.broadcasted_iota(jnp.int32, d2s.shape, 0) + i * _BR
    d2s = jnp.where((col == row) | (col >= _N_KEEP), jnp.inf, d2s)
    sel_ref[...] = d2s

    kcol = jax.lax.broadcasted_iota(jnp.int32, (_BR, _KPAD), 1)

    def body(t, carry):
        prev_m, prev_idx, nbr_acc = carry
        v0 = sel_ref[...]
        v = jnp.where((v0 == prev_m) & (col == prev_idx), jnp.inf, v0)
        sel_ref[...] = v
        m = jnp.min(v, axis=1, keepdims=True)
        idx = jnp.min(jnp.where(sel_ref[...] == m, col, _PAD), axis=1,
                      keepdims=True)
        nbr_acc = jnp.where(kcol == t, idx, nbr_acc)
        return m, idx, nbr_acc

    m0 = jnp.full((_BR, 1), -jnp.inf, jnp.float32)
    i0 = jnp.full((_BR, 1), -1, jnp.int32)
    _, _, nbr_acc = jax.lax.fori_loop(
        0, _K, body, (m0, i0, jnp.zeros((_BR, _KPAD), jnp.int32)))
    nbr_ref[...] = nbr_acc


_sc_mesh = plsc.VectorSubcoreMesh(core_axis_name="c", subcore_axis_name="s")


@functools.partial(
    pl.kernel, mesh=_sc_mesh,
    out_type=jax.ShapeDtypeStruct((_EPAD,), jnp.float32),
    compiler_params=pltpu.CompilerParams(needs_layout_passes=False),
    scratch_types=[
        pltpu.VMEM((_PAD,), jnp.float32),
        pltpu.VMEM((_PAD,), jnp.float32),
        pltpu.VMEM((_PAD,), jnp.float32),
        pltpu.VMEM((_BPW,), jnp.int32),
        pltpu.VMEM((_BPW,), jnp.int32),
        pltpu.VMEM((_BPW,), jnp.float32),
    ],
)
def _sc_edge_d2(x_hbm, y_hbm, z_hbm, is_hbm, id_hbm, out_hbm,
                xv, yv, zv, isv, idv, ov):
    wid = lax.axis_index("s") * 2 + lax.axis_index("c")
    base = wid * _BPW
    pltpu.sync_copy(x_hbm, xv)
    pltpu.sync_copy(y_hbm, yv)
    pltpu.sync_copy(z_hbm, zv)
    pltpu.sync_copy(is_hbm.at[pl.ds(base, _BPW)], isv)
    pltpu.sync_copy(id_hbm.at[pl.ds(base, _BPW)], idv)

    def body(g, _):
        off = g * 16
        ii = isv[pl.ds(off, 16)]
        jj = idv[pl.ds(off, 16)]
        dx = plsc.load_gather(xv, [ii]) - plsc.load_gather(xv, [jj])
        dy = plsc.load_gather(yv, [ii]) - plsc.load_gather(yv, [jj])
        dz = plsc.load_gather(zv, [ii]) - plsc.load_gather(zv, [jj])
        ov[pl.ds(off, 16)] = (dx * dx + dy * dy) + dz * dz
        return 0

    lax.fori_loop(0, _BPW // 16, body, 0)
    pltpu.sync_copy(ov, out_hbm.at[pl.ds(base, _BPW)])


def _rbf_kernel(d2_ref, *out_refs):
    dist = jnp.sqrt(d2_ref[...] + 1e-12)
    sigma = _CUTOFF / (_NUM_BINS - 1)
    for b in range(_NUM_BINS):
        out_refs[b][...] = jnp.exp(
            -((dist - b * sigma) ** 2) / (2.0 * sigma * sigma))


@jax.jit
def kernel(pos):
    keep_idx, noise = _augment_consts()
    p = jnp.take(pos, keep_idx, axis=0) + noise

    p_pad = jnp.zeros((_PAD, _F), jnp.float32).at[:_N_KEEP, :3].set(p)
    btb = p_pad.T.astype(jnp.bfloat16)
    sqc = jnp.broadcast_to(jnp.sum(p_pad * p_pad, axis=1)[None, :], (8, _PAD))

    nbr_full = pl.pallas_call(
        _knn_kernel,
        grid=(_PAD // _BR,),
        in_specs=[
            pl.BlockSpec((_BR, _F), lambda i: (i, 0)),
            pl.BlockSpec((_F, _PAD), lambda i: (0, 0)),
            pl.BlockSpec((8, _PAD), lambda i: (0, 0)),
        ],
        out_specs=pl.BlockSpec((_BR, _KPAD), lambda i: (i, 0)),
        out_shape=jax.ShapeDtypeStruct((_PAD, _KPAD), jnp.int32),
        scratch_shapes=[pltpu.VMEM((_BR, _PAD), jnp.float32)],
    )(p_pad, btb, sqc)

    nbr = nbr_full[:_N_KEEP, :_K]
    src = nbr.reshape(-1)
    dst = jnp.repeat(jnp.arange(_N_KEEP, dtype=src.dtype), _K)

    src_pad = jnp.zeros((_EPAD,), jnp.int32).at[:_E].set(src)
    dst_pad = jnp.zeros((_EPAD,), jnp.int32).at[:_E].set(dst)
    d2_edges = _sc_edge_d2(p_pad[:, 0], p_pad[:, 1], p_pad[:, 2],
                           src_pad, dst_pad)

    bins = pl.pallas_call(
        _rbf_kernel,
        out_shape=[jax.ShapeDtypeStruct((_EPAD // 128, 128), jnp.float32)
                   for _ in range(_NUM_BINS)],
    )(d2_edges.reshape(_EPAD // 128, 128))
    ea = jnp.stack([b.reshape(-1) for b in bins], axis=1)[:_E]
    edge_attr = jnp.concatenate([ea, ea], axis=0)

    edge_index = jnp.stack(
        [jnp.concatenate([src, dst]), jnp.concatenate([dst, src])], axis=0)
    return edge_index, edge_attr


# speedup vs baseline: 1.0090x; 1.0090x over previous
import functools

import jax
import jax.numpy as jnp
from jax import lax
from jax.experimental import pallas as pl
from jax.experimental.pallas import tpu as pltpu, tpu_sc as plsc

_N = 10000
_NODE_MASKING = 0.3
_RADIUS = 0.75
_K = 50
_NUM_BINS = 5
_CUTOFF = 5.0
_N_KEEP = int(_N * (1.0 - _NODE_MASKING))

_BR = 512
_PAD = 7168
_KPAD = 64
_F = 128

_E = _N_KEEP * _K
_NW = 32
_EPAD = 350208
_BPW = _EPAD // _NW


def _augment_consts():
    base = jax.random.key(1)
    k1 = jax.random.fold_in(base, 0)
    k2 = jax.random.fold_in(base, 1)
    k3 = jax.random.fold_in(base, 2)
    scores = jax.random.uniform(k1, (_N,))
    keep_idx = jnp.argsort(scores)[:_N_KEEP]
    dirs = jax.random.normal(k2, (_N_KEEP, 3), dtype=jnp.float32)
    dirs = dirs / (jnp.linalg.norm(dirs, axis=1, keepdims=True) + 1e-12)
    u = jax.random.uniform(k3, (_N_KEEP, 1), dtype=jnp.float32)
    noise = dirs * _RADIUS * (u ** (1.0 / 3.0))
    return keep_idx, noise


def _knn_kernel(a_ref, btb_ref, sqc_ref, nbr_ref, sel_ref):
    i = pl.program_id(0)
    a = a_ref[...]
    sq_r = jnp.sum(a * a, axis=1, keepdims=True)
    dot = jnp.dot(a.astype(jnp.bfloat16), btb_ref[...],
                  preferred_element_type=jnp.float32)
    d2s = sq_r + sqc_ref[0:1, :] - 2.0 * dot
    col = jax.lax.broadcasted_iota(jnp.int32, d2s.shape, 1)
    row = jax.lax.broadcasted_iota(jnp.int32, d2s.shape, 0) + i * _BR
    d2s = jnp.where((col == row) | (col >= _N_KEEP), jnp.inf, d2s)
    sel_ref[...] = d2s

    kcol = jax.lax.broadcasted_iota(jnp.int32, (_BR, _KPAD), 1)

    def body(t, carry):
        prev_m, prev_idx, nbr_acc = carry
        v0 = sel_ref[...]
        v = jnp.where((v0 == prev_m) & (col == prev_idx), jnp.inf, v0)
        sel_ref[...] = v
        m = jnp.min(v, axis=1, keepdims=True)
        idx = jnp.min(jnp.where(v == m, col, _PAD), axis=1,
                      keepdims=True)
        nbr_acc = jnp.where(kcol == t, idx, nbr_acc)
        return m, idx, nbr_acc

    m0 = jnp.full((_BR, 1), -jnp.inf, jnp.float32)
    i0 = jnp.full((_BR, 1), -1, jnp.int32)
    _, _, nbr_acc = jax.lax.fori_loop(
        0, _K, body, (m0, i0, jnp.zeros((_BR, _KPAD), jnp.int32)))
    nbr_ref[...] = nbr_acc


_sc_mesh = plsc.VectorSubcoreMesh(core_axis_name="c", subcore_axis_name="s")


@functools.partial(
    pl.kernel, mesh=_sc_mesh,
    out_type=jax.ShapeDtypeStruct((_EPAD,), jnp.float32),
    compiler_params=pltpu.CompilerParams(needs_layout_passes=False),
    scratch_types=[
        pltpu.VMEM((_PAD,), jnp.float32),
        pltpu.VMEM((_PAD,), jnp.float32),
        pltpu.VMEM((_PAD,), jnp.float32),
        pltpu.VMEM((_BPW,), jnp.int32),
        pltpu.VMEM((_BPW,), jnp.int32),
        pltpu.VMEM((_BPW,), jnp.float32),
    ],
)
def _sc_edge_d2(x_hbm, y_hbm, z_hbm, is_hbm, id_hbm, out_hbm,
                xv, yv, zv, isv, idv, ov):
    wid = lax.axis_index("s") * 2 + lax.axis_index("c")
    base = wid * _BPW
    pltpu.sync_copy(x_hbm, xv)
    pltpu.sync_copy(y_hbm, yv)
    pltpu.sync_copy(z_hbm, zv)
    pltpu.sync_copy(is_hbm.at[pl.ds(base, _BPW)], isv)
    pltpu.sync_copy(id_hbm.at[pl.ds(base, _BPW)], idv)

    def body(g, _):
        off = g * 16
        ii = isv[pl.ds(off, 16)]
        jj = idv[pl.ds(off, 16)]
        dx = plsc.load_gather(xv, [ii]) - plsc.load_gather(xv, [jj])
        dy = plsc.load_gather(yv, [ii]) - plsc.load_gather(yv, [jj])
        dz = plsc.load_gather(zv, [ii]) - plsc.load_gather(zv, [jj])
        ov[pl.ds(off, 16)] = (dx * dx + dy * dy) + dz * dz
        return 0

    lax.fori_loop(0, _BPW // 16, body, 0)
    pltpu.sync_copy(ov, out_hbm.at[pl.ds(base, _BPW)])


def _rbf_kernel(d2_ref, *out_refs):
    dist = jnp.sqrt(d2_ref[...] + 1e-12)
    sigma = _CUTOFF / (_NUM_BINS - 1)
    for b in range(_NUM_BINS):
        out_refs[b][...] = jnp.exp(
            -((dist - b * sigma) ** 2) / (2.0 * sigma * sigma))


@jax.jit
def kernel(pos):
    keep_idx, noise = _augment_consts()
    p = jnp.take(pos, keep_idx, axis=0) + noise

    p_pad = jnp.zeros((_PAD, _F), jnp.float32).at[:_N_KEEP, :3].set(p)
    btb = p_pad.T.astype(jnp.bfloat16)
    sqc = jnp.broadcast_to(jnp.sum(p_pad * p_pad, axis=1)[None, :], (8, _PAD))

    nbr_full = pl.pallas_call(
        _knn_kernel,
        grid=(_PAD // _BR,),
        in_specs=[
            pl.BlockSpec((_BR, _F), lambda i: (i, 0)),
            pl.BlockSpec((_F, _PAD), lambda i: (0, 0)),
            pl.BlockSpec((8, _PAD), lambda i: (0, 0)),
        ],
        out_specs=pl.BlockSpec((_BR, _KPAD), lambda i: (i, 0)),
        out_shape=jax.ShapeDtypeStruct((_PAD, _KPAD), jnp.int32),
        scratch_shapes=[pltpu.VMEM((_BR, _PAD), jnp.float32)],
    )(p_pad, btb, sqc)

    nbr = nbr_full[:_N_KEEP, :_K]
    src = nbr.reshape(-1)
    dst = jnp.repeat(jnp.arange(_N_KEEP, dtype=src.dtype), _K)

    src_pad = jnp.zeros((_EPAD,), jnp.int32).at[:_E].set(src)
    dst_pad = jnp.zeros((_EPAD,), jnp.int32).at[:_E].set(dst)
    d2_edges = _sc_edge_d2(p_pad[:, 0], p_pad[:, 1], p_pad[:, 2],
                           src_pad, dst_pad)

    bins = pl.pallas_call(
        _rbf_kernel,
        out_shape=[jax.ShapeDtypeStruct((_EPAD // 128, 128), jnp.float32)
                   for _ in range(_NUM_BINS)],
    )(d2_edges.reshape(_EPAD // 128, 128))
    ea = jnp.stack([b.reshape(-1) for b in bins], axis=1)[:_E]
    edge_attr = jnp.concatenate([ea, ea], axis=0)

    edge_index = jnp.stack(
        [jnp.concatenate([src, dst]), jnp.concatenate([dst, src])], axis=0)
    return edge_index, edge_attr
